# baseline (speedup 1.0000x reference)
import functools

import jax
import jax.numpy as jnp
from jax.experimental import pallas as pl
from jax.experimental.pallas import tpu as pltpu

NUM_EMBEDDINGS = 1024
EMBEDDING_DIM = 64
BETA = 0.25
B = 16
P = 1024
IPS = 2
N_TOTAL = B * P
N_STEPS = B // IPS


def _vq_kernel(zb_ref, emb_ref, z2_ref, e2_ref,
               loss_ref, zq_ref, perp_ref, enc_ref, idx_ref,
               counts_ref, sse_ref):
    g = pl.program_id(0)
    K = NUM_EMBEDDINGS

    emb = emb_ref[...]
    e2 = e2_ref[...]
    emb2 = emb * 2.0

    iota_col = jax.lax.broadcasted_iota(jnp.int32, (K, 1), 0)
    hi_col = (iota_col // 32).astype(jnp.float32)
    lo_col = (iota_col % 32).astype(jnp.float32)
    emb_aug = jnp.concatenate(
        [emb, hi_col, lo_col, jnp.ones((K, 1), jnp.float32)], axis=1)

    tot_counts = jnp.zeros((1, K), jnp.float32)
    tot_sse = jnp.float32(0.0)
    fixups = []

    for i in range(IPS):
        zb = zb_ref[0, i * EMBEDDING_DIM:(i + 1) * EMBEDDING_DIM, :]
        z2 = z2_ref[0, i * P:(i + 1) * P, :]

        m2 = jax.lax.dot_general(
            zb, emb2, dimension_numbers=(((0,), (1,)), ((), ())),
            preferred_element_type=jnp.float32)
        d = (z2 + e2) - m2

        minv = jnp.min(d, axis=1, keepdims=True)
        multi = jnp.where(d == minv, 1.0, 0.0).astype(jnp.float32)
        enc_ref[i * P:(i + 1) * P, :] = multi

        aug = jax.lax.dot_general(
            emb_aug, multi, dimension_numbers=(((0,), (1,)), ((), ())),
            preferred_element_type=jnp.float32)
        zq_t = aug[0:EMBEDDING_DIM]
        zq_ref[0, i * EMBEDDING_DIM:(i + 1) * EMBEDDING_DIM, :] = (
            zb + (zq_t - zb))
        idx_row = (32.0 * aug[EMBEDDING_DIM:EMBEDDING_DIM + 1]
                   + aug[EMBEDDING_DIM + 1:EMBEDDING_DIM + 2]
                   ).astype(jnp.int32)
        idx_ref[i * P:(i + 1) * P, :] = jnp.transpose(idx_row, (1, 0))
        has_tie = jnp.max(aug[EMBEDDING_DIM + 2]) > 1.0

        ones_row = jnp.ones((1, P), jnp.float32)
        part_counts = jax.lax.dot_general(
            ones_row, multi, dimension_numbers=(((1,), (0,)), ((), ())),
            preferred_element_type=jnp.float32)
        diff = zq_t - zb
        part_sse = jnp.sum(diff * diff)
        tot_counts = tot_counts + part_counts
        tot_sse = tot_sse + part_sse
        fixups.append((has_tie, zb, d, minv, i, part_counts, part_sse))

    @pl.when(g == 0)
    def _init():
        sse_ref[0, 0] = tot_sse
        counts_ref[...] = tot_counts

    @pl.when(g > 0)
    def _acc():
        sse_ref[0, 0] += tot_sse
        counts_ref[...] += tot_counts

    for has_tie, zb, d, minv, i, part_counts, part_sse in fixups:
        @pl.when(has_tie)
        def _fix_ties(zb=zb, d=d, minv=minv, i=i,
                      part_counts=part_counts, part_sse=part_sse):
            iota_k = jax.lax.broadcasted_iota(jnp.int32, (P, K), 1)
            masked = jnp.where(d == minv, iota_k, K)
            idxc = jnp.min(masked, axis=1, keepdims=True)
            one_hot = (iota_k == idxc).astype(jnp.float32)
            enc_ref[i * P:(i + 1) * P, :] = one_hot
            idx_ref[i * P:(i + 1) * P, :] = idxc
            aug2 = jax.lax.dot_general(
                emb_aug, one_hot, dimension_numbers=(((0,), (1,)), ((), ())),
                preferred_element_type=jnp.float32)
            zq2 = aug2[0:EMBEDDING_DIM]
            zq_ref[0, i * EMBEDDING_DIM:(i + 1) * EMBEDDING_DIM, :] = (
                zb + (zq2 - zb))
            cnt2 = jax.lax.dot_general(
                jnp.ones((1, P), jnp.float32), one_hot,
                dimension_numbers=(((1,), (0,)), ((), ())),
                preferred_element_type=jnp.float32)
            diff2 = zq2 - zb
            counts_ref[...] += cnt2 - part_counts
            sse_ref[0, 0] += jnp.sum(diff2 * diff2) - part_sse

    @pl.when(g == N_STEPS - 1)
    def _finalize():
        sse = sse_ref[0, 0]
        loss_ref[...] = jnp.reshape(
            (1.0 + BETA) * sse / float(N_TOTAL * EMBEDDING_DIM), (1, 1))
        me = counts_ref[...] / float(N_TOTAL)
        perp_ref[...] = jnp.reshape(
            jnp.exp(-jnp.sum(me + jnp.log(me + 1e-10))), (1, 1))


@functools.partial(jax.jit, static_argnames=())
def kernel(z, embedding):
    zp = jnp.transpose(z, (0, 2, 3, 1))
    z_flat = zp.reshape(-1, EMBEDDING_DIM)
    z2 = jnp.sum(z_flat ** 2, axis=1)
    e2 = jnp.sum(embedding ** 2, axis=1)

    z_cp = z.reshape(N_STEPS, IPS * EMBEDDING_DIM, P)
    z2_r = z2.reshape(N_STEPS, IPS * P, 1)
    e2_r = e2.reshape(1, NUM_EMBEDDINGS)

    grid = (N_STEPS,)
    out_shapes = (
        jax.ShapeDtypeStruct((1, 1), jnp.float32),
        jax.ShapeDtypeStruct((N_STEPS, IPS * EMBEDDING_DIM, P), jnp.float32),
        jax.ShapeDtypeStruct((1, 1), jnp.float32),
        jax.ShapeDtypeStruct((N_TOTAL, NUM_EMBEDDINGS), jnp.float32),
        jax.ShapeDtypeStruct((N_TOTAL, 1), jnp.int32),
    )
    in_specs = [
        pl.BlockSpec((1, IPS * EMBEDDING_DIM, P), lambda g: (g, 0, 0)),
        pl.BlockSpec((NUM_EMBEDDINGS, EMBEDDING_DIM), lambda g: (0, 0)),
        pl.BlockSpec((1, IPS * P, 1), lambda g: (g, 0, 0)),
        pl.BlockSpec((1, NUM_EMBEDDINGS), lambda g: (0, 0)),
    ]
    out_specs = (
        pl.BlockSpec((1, 1), lambda g: (0, 0)),
        pl.BlockSpec((1, IPS * EMBEDDING_DIM, P), lambda g: (g, 0, 0)),
        pl.BlockSpec((1, 1), lambda g: (0, 0)),
        pl.BlockSpec((IPS * P, NUM_EMBEDDINGS), lambda g: (g, 0)),
        pl.BlockSpec((IPS * P, 1), lambda g: (g, 0)),
    )
    loss, zq, perp, enc, idx = pl.pallas_call(
        _vq_kernel,
        grid=grid,
        in_specs=in_specs,
        out_specs=out_specs,
        out_shape=out_shapes,
        scratch_shapes=[
            pltpu.VMEM((1, NUM_EMBEDDINGS), jnp.float32),
            pltpu.SMEM((1, 1), jnp.float32),
        ],
    )(z_cp, embedding, z2_r, e2_r)

    z_q_out = zq.reshape(z.shape)
    return (loss[0, 0], z_q_out, perp[0, 0], enc, idx)


# speedup vs baseline: 1.3768x; 1.3768x over previous
import functools

import jax
import jax.numpy as jnp
from jax.experimental import pallas as pl
from jax.experimental.pallas import tpu as pltpu

NUM_EMBEDDINGS = 1024
EMBEDDING_DIM = 64
BETA = 0.25
B = 16
P = 1024
IPS = 1
N_TOTAL = B * P
N_STEPS = B // IPS


def _vq_kernel(zb_ref, emb_ref, z2_ref, e2_ref,
               loss_ref, zq_ref, perp_ref, enc_ref, idx_ref,
               counts_ref, sse_ref):
    g = pl.program_id(0)
    K = NUM_EMBEDDINGS

    emb = emb_ref[...]
    e2 = e2_ref[...]
    emb2 = emb * 2.0

    iota_col = jax.lax.broadcasted_iota(jnp.int32, (K, 1), 0)
    hi_col = (iota_col // 32).astype(jnp.float32)
    lo_col = (iota_col % 32).astype(jnp.float32)
    emb_aug = jnp.concatenate(
        [emb, hi_col, lo_col, jnp.ones((K, 1), jnp.float32)], axis=1)

    tot_counts = jnp.zeros((1, K), jnp.float32)
    tot_sse = jnp.float32(0.0)
    fixups = []

    for i in range(IPS):
        zb = zb_ref[0, i * EMBEDDING_DIM:(i + 1) * EMBEDDING_DIM, :]
        z2 = z2_ref[0, i * P:(i + 1) * P, :]

        m2 = jax.lax.dot_general(
            zb, emb2, dimension_numbers=(((0,), (1,)), ((), ())),
            preferred_element_type=jnp.float32)
        d = (z2 + e2) - m2

        minv = jnp.min(d, axis=1, keepdims=True)
        multi = jnp.where(d == minv, 1.0, 0.0).astype(jnp.float32)
        enc_ref[i * P:(i + 1) * P, :] = multi

        aug = jax.lax.dot_general(
            emb_aug, multi, dimension_numbers=(((0,), (1,)), ((), ())),
            preferred_element_type=jnp.float32)
        zq_t = aug[0:EMBEDDING_DIM]
        zq_ref[0, i * EMBEDDING_DIM:(i + 1) * EMBEDDING_DIM, :] = (
            zb + (zq_t - zb))
        idx_row = (32.0 * aug[EMBEDDING_DIM:EMBEDDING_DIM + 1]
                   + aug[EMBEDDING_DIM + 1:EMBEDDING_DIM + 2]
                   ).astype(jnp.int32)
        idx_ref[i * P:(i + 1) * P, :] = jnp.transpose(idx_row, (1, 0))
        has_tie = jnp.max(aug[EMBEDDING_DIM + 2]) > 1.0

        ones_row = jnp.ones((1, P), jnp.float32)
        part_counts = jax.lax.dot_general(
            ones_row, multi, dimension_numbers=(((1,), (0,)), ((), ())),
            preferred_element_type=jnp.float32)
        diff = zq_t - zb
        part_sse = jnp.sum(diff * diff)
        tot_counts = tot_counts + part_counts
        tot_sse = tot_sse + part_sse
        fixups.append((has_tie, zb, d, minv, i, part_counts, part_sse))

    @pl.when(g == 0)
    def _init():
        sse_ref[0, 0] = tot_sse
        counts_ref[...] = tot_counts

    @pl.when(g > 0)
    def _acc():
        sse_ref[0, 0] += tot_sse
        counts_ref[...] += tot_counts

    for has_tie, zb, d, minv, i, part_counts, part_sse in fixups:
        @pl.when(has_tie)
        def _fix_ties(zb=zb, d=d, minv=minv, i=i,
                      part_counts=part_counts, part_sse=part_sse):
            iota_k = jax.lax.broadcasted_iota(jnp.int32, (P, K), 1)
            masked = jnp.where(d == minv, iota_k, K)
            idxc = jnp.min(masked, axis=1, keepdims=True)
            one_hot = (iota_k == idxc).astype(jnp.float32)
            enc_ref[i * P:(i + 1) * P, :] = one_hot
            idx_ref[i * P:(i + 1) * P, :] = idxc
            aug2 = jax.lax.dot_general(
                emb_aug, one_hot, dimension_numbers=(((0,), (1,)), ((), ())),
                preferred_element_type=jnp.float32)
            zq2 = aug2[0:EMBEDDING_DIM]
            zq_ref[0, i * EMBEDDING_DIM:(i + 1) * EMBEDDING_DIM, :] = (
                zb + (zq2 - zb))
            cnt2 = jax.lax.dot_general(
                jnp.ones((1, P), jnp.float32), one_hot,
                dimension_numbers=(((1,), (0,)), ((), ())),
                preferred_element_type=jnp.float32)
            diff2 = zq2 - zb
            counts_ref[...] += cnt2 - part_counts
            sse_ref[0, 0] += jnp.sum(diff2 * diff2) - part_sse

    @pl.when(g == N_STEPS - 1)
    def _finalize():
        sse = sse_ref[0, 0]
        loss_ref[...] = jnp.reshape(
            (1.0 + BETA) * sse / float(N_TOTAL * EMBEDDING_DIM), (1, 1))
        me = counts_ref[...] / float(N_TOTAL)
        perp_ref[...] = jnp.reshape(
            jnp.exp(-jnp.sum(me + jnp.log(me + 1e-10))), (1, 1))


@functools.partial(jax.jit, static_argnames=())
def kernel(z, embedding):
    zp = jnp.transpose(z, (0, 2, 3, 1))
    z_flat = zp.reshape(-1, EMBEDDING_DIM)
    z2 = jnp.sum(z_flat ** 2, axis=1)
    e2 = jnp.sum(embedding ** 2, axis=1)

    z_cp = z.reshape(N_STEPS, IPS * EMBEDDING_DIM, P)
    z2_r = z2.reshape(N_STEPS, IPS * P, 1)
    e2_r = e2.reshape(1, NUM_EMBEDDINGS)

    grid = (N_STEPS,)
    out_shapes = (
        jax.ShapeDtypeStruct((1, 1), jnp.float32),
        jax.ShapeDtypeStruct((N_STEPS, IPS * EMBEDDING_DIM, P), jnp.float32),
        jax.ShapeDtypeStruct((1, 1), jnp.float32),
        jax.ShapeDtypeStruct((N_TOTAL, NUM_EMBEDDINGS), jnp.float32),
        jax.ShapeDtypeStruct((N_TOTAL, 1), jnp.int32),
    )
    in_specs = [
        pl.BlockSpec((1, IPS * EMBEDDING_DIM, P), lambda g: (g, 0, 0)),
        pl.BlockSpec((NUM_EMBEDDINGS, EMBEDDING_DIM), lambda g: (0, 0)),
        pl.BlockSpec((1, IPS * P, 1), lambda g: (g, 0, 0)),
        pl.BlockSpec((1, NUM_EMBEDDINGS), lambda g: (0, 0)),
    ]
    out_specs = (
        pl.BlockSpec((1, 1), lambda g: (0, 0)),
        pl.BlockSpec((1, IPS * EMBEDDING_DIM, P), lambda g: (g, 0, 0)),
        pl.BlockSpec((1, 1), lambda g: (0, 0)),
        pl.BlockSpec((IPS * P, NUM_EMBEDDINGS), lambda g: (g, 0)),
        pl.BlockSpec((IPS * P, 1), lambda g: (g, 0)),
    )
    loss, zq, perp, enc, idx = pl.pallas_call(
        _vq_kernel,
        grid=grid,
        in_specs=in_specs,
        out_specs=out_specs,
        out_shape=out_shapes,
        scratch_shapes=[
            pltpu.VMEM((1, NUM_EMBEDDINGS), jnp.float32),
            pltpu.SMEM((1, 1), jnp.float32),
        ],
    )(z_cp, embedding, z2_r, e2_r)

    z_q_out = zq.reshape(z.shape)
    return (loss[0, 0], z_q_out, perp[0, 0], enc, idx)


# speedup vs baseline: 1.3863x; 1.0069x over previous
import functools

import jax
import jax.numpy as jnp
from jax.experimental import pallas as pl
from jax.experimental.pallas import tpu as pltpu

NUM_EMBEDDINGS = 1024
EMBEDDING_DIM = 64
BETA = 0.25
B = 16
P = 1024
IPS = 1
N_TOTAL = B * P
N_STEPS = B // IPS


def _vq_kernel(zb_ref, emb_ref, z2_ref, e2_ref,
               loss_ref, zq_ref, perp_ref, enc_ref, idx_ref,
               counts_ref, sse_ref, emb2_ref, embaug_ref):
    g = pl.program_id(0)
    K = NUM_EMBEDDINGS

    e2 = e2_ref[...]

    @pl.when(g == 0)
    def _prep():
        emb = emb_ref[...]
        emb2_ref[...] = emb * 2.0
        iota_col = jax.lax.broadcasted_iota(jnp.int32, (K, 1), 0)
        hi_col = (iota_col // 32).astype(jnp.float32)
        lo_col = (iota_col % 32).astype(jnp.float32)
        embaug_ref[...] = jnp.concatenate(
            [emb, hi_col, lo_col, jnp.ones((K, 1), jnp.float32)], axis=1)

    emb2 = emb2_ref[...]
    emb_aug = embaug_ref[...]

    tot_counts = jnp.zeros((1, K), jnp.float32)
    tot_sse = jnp.float32(0.0)
    fixups = []

    for i in range(IPS):
        zb = zb_ref[0, i * EMBEDDING_DIM:(i + 1) * EMBEDDING_DIM, :]
        z2 = z2_ref[0, i * P:(i + 1) * P, :]

        m2 = jax.lax.dot_general(
            zb, emb2, dimension_numbers=(((0,), (1,)), ((), ())),
            preferred_element_type=jnp.float32)
        d = (z2 + e2) - m2

        minv = jnp.min(d, axis=1, keepdims=True)
        multi = jnp.where(d == minv, 1.0, 0.0).astype(jnp.float32)
        enc_ref[i * P:(i + 1) * P, :] = multi

        aug = jax.lax.dot_general(
            emb_aug, multi, dimension_numbers=(((0,), (1,)), ((), ())),
            preferred_element_type=jnp.float32)
        zq_t = aug[0:EMBEDDING_DIM]
        zq_ref[0, i * EMBEDDING_DIM:(i + 1) * EMBEDDING_DIM, :] = (
            zb + (zq_t - zb))
        idx_row = (32.0 * aug[EMBEDDING_DIM:EMBEDDING_DIM + 1]
                   + aug[EMBEDDING_DIM + 1:EMBEDDING_DIM + 2]
                   ).astype(jnp.int32)
        idx_ref[i * P:(i + 1) * P, :] = jnp.transpose(idx_row, (1, 0))
        has_tie = jnp.max(aug[EMBEDDING_DIM + 2]) > 1.0

        ones_row = jnp.ones((1, P), jnp.float32)
        part_counts = jax.lax.dot_general(
            ones_row, multi, dimension_numbers=(((1,), (0,)), ((), ())),
            preferred_element_type=jnp.float32)
        diff = zq_t - zb
        part_sse = jnp.sum(diff * diff)
        tot_counts = tot_counts + part_counts
        tot_sse = tot_sse + part_sse
        fixups.append((has_tie, zb, d, minv, i, part_counts, part_sse))

    @pl.when(g == 0)
    def _init():
        sse_ref[0, 0] = tot_sse
        counts_ref[...] = tot_counts

    @pl.when(g > 0)
    def _acc():
        sse_ref[0, 0] += tot_sse
        counts_ref[...] += tot_counts

    for has_tie, zb, d, minv, i, part_counts, part_sse in fixups:
        @pl.when(has_tie)
        def _fix_ties(zb=zb, d=d, minv=minv, i=i,
                      part_counts=part_counts, part_sse=part_sse):
            iota_k = jax.lax.broadcasted_iota(jnp.int32, (P, K), 1)
            masked = jnp.where(d == minv, iota_k, K)
            idxc = jnp.min(masked, axis=1, keepdims=True)
            one_hot = (iota_k == idxc).astype(jnp.float32)
            enc_ref[i * P:(i + 1) * P, :] = one_hot
            idx_ref[i * P:(i + 1) * P, :] = idxc
            aug2 = jax.lax.dot_general(
                emb_aug, one_hot, dimension_numbers=(((0,), (1,)), ((), ())),
                preferred_element_type=jnp.float32)
            zq2 = aug2[0:EMBEDDING_DIM]
            zq_ref[0, i * EMBEDDING_DIM:(i + 1) * EMBEDDING_DIM, :] = (
                zb + (zq2 - zb))
            cnt2 = jax.lax.dot_general(
                jnp.ones((1, P), jnp.float32), one_hot,
                dimension_numbers=(((1,), (0,)), ((), ())),
                preferred_element_type=jnp.float32)
            diff2 = zq2 - zb
            counts_ref[...] += cnt2 - part_counts
            sse_ref[0, 0] += jnp.sum(diff2 * diff2) - part_sse

    @pl.when(g == N_STEPS - 1)
    def _finalize():
        sse = sse_ref[0, 0]
        loss_ref[...] = jnp.reshape(
            (1.0 + BETA) * sse / float(N_TOTAL * EMBEDDING_DIM), (1, 1))
        me = counts_ref[...] / float(N_TOTAL)
        perp_ref[...] = jnp.reshape(
            jnp.exp(-jnp.sum(me + jnp.log(me + 1e-10))), (1, 1))


@functools.partial(jax.jit, static_argnames=())
def kernel(z, embedding):
    zp = jnp.transpose(z, (0, 2, 3, 1))
    z_flat = zp.reshape(-1, EMBEDDING_DIM)
    z2 = jnp.sum(z_flat ** 2, axis=1)
    e2 = jnp.sum(embedding ** 2, axis=1)

    z_cp = z.reshape(N_STEPS, IPS * EMBEDDING_DIM, P)
    z2_r = z2.reshape(N_STEPS, IPS * P, 1)
    e2_r = e2.reshape(1, NUM_EMBEDDINGS)

    grid = (N_STEPS,)
    out_shapes = (
        jax.ShapeDtypeStruct((1, 1), jnp.float32),
        jax.ShapeDtypeStruct((N_STEPS, IPS * EMBEDDING_DIM, P), jnp.float32),
        jax.ShapeDtypeStruct((1, 1), jnp.float32),
        jax.ShapeDtypeStruct((N_TOTAL, NUM_EMBEDDINGS), jnp.float32),
        jax.ShapeDtypeStruct((N_TOTAL, 1), jnp.int32),
    )
    in_specs = [
        pl.BlockSpec((1, IPS * EMBEDDING_DIM, P), lambda g: (g, 0, 0)),
        pl.BlockSpec((NUM_EMBEDDINGS, EMBEDDING_DIM), lambda g: (0, 0)),
        pl.BlockSpec((1, IPS * P, 1), lambda g: (g, 0, 0)),
        pl.BlockSpec((1, NUM_EMBEDDINGS), lambda g: (0, 0)),
    ]
    out_specs = (
        pl.BlockSpec((1, 1), lambda g: (0, 0)),
        pl.BlockSpec((1, IPS * EMBEDDING_DIM, P), lambda g: (g, 0, 0)),
        pl.BlockSpec((1, 1), lambda g: (0, 0)),
        pl.BlockSpec((IPS * P, NUM_EMBEDDINGS), lambda g: (g, 0)),
        pl.BlockSpec((IPS * P, 1), lambda g: (g, 0)),
    )
    loss, zq, perp, enc, idx = pl.pallas_call(
        _vq_kernel,
        grid=grid,
        in_specs=in_specs,
        out_specs=out_specs,
        out_shape=out_shapes,
        scratch_shapes=[
            pltpu.VMEM((1, NUM_EMBEDDINGS), jnp.float32),
            pltpu.SMEM((1, 1), jnp.float32),
            pltpu.VMEM((NUM_EMBEDDINGS, EMBEDDING_DIM), jnp.float32),
            pltpu.VMEM((NUM_EMBEDDINGS, EMBEDDING_DIM + 3), jnp.float32),
        ],
    )(z_cp, embedding, z2_r, e2_r)

    z_q_out = zq.reshape(z.shape)
    return (loss[0, 0], z_q_out, perp[0, 0], enc, idx)
